# baseline (speedup 1.0000x reference)
import functools

import jax
import jax.numpy as jnp
from jax import lax
from jax.experimental import pallas as pl
from jax.experimental.pallas import tpu as pltpu
from jax.experimental.pallas import tpu_sc as plsc

N = 10000
E = 320000
D = 128
R = 20
NB = 4
BLK = 32
L = 100
BD = 1024
LE = 104

NW = 32
EPT = E // NW
EC = 80
ECH = EPT // EC
NPT = 624
NTAIL = N - 16 * NPT

WG = BD * LE
WPT = WG // NW
WC = 64
WCH = WPT // WC

_mesh = lambda: plsc.VectorSubcoreMesh(core_axis_name="c", subcore_axis_name="s")


def _edge_agg_body(tx_hbm, k_hbm, d_hbm, n_hbm, out_hbm,
                   kv0, kv1, kv2, dv0, dv1, dv2, nv0, nv1, nv2,
                   rows0, rows1, rows2, acc,
                   si0, si1, si2, sg0, sg1, sg2, ss0, ss1, ss2):
    c = lax.axis_index("c")
    s = lax.axis_index("s")
    wid = s * 2 + c
    KV = (kv0, kv1, kv2)
    DV = (dv0, dv1, dv2)
    NV = (nv0, nv1, nv2)
    ROWS = (rows0, rows1, rows2)
    SI = (si0, si1, si2)
    SG = (sg0, sg1, sg2)
    SS = (ss0, ss1, ss2)

    z16 = jnp.zeros((16,), jnp.float32)
    for j in range(8):
        def zrow(i, _, j=j):
            rows0[i, pl.ds(j * 16, 16)] = z16
            return 0
        lax.fori_loop(0, EC, zrow, 0)

    def zcp(i, _):
        pltpu.sync_copy(rows0, acc.at[pl.ds(s * NPT + i * EC, EC)])
        return 0
    lax.fori_loop(0, 7, zcp, 0)
    pltpu.sync_copy(rows0.at[pl.ds(0, 64)],
                    acc.at[pl.ds(s * NPT + 7 * EC, 64)])

    @pl.when(s == 15)
    def _():
        pltpu.sync_copy(rows0.at[pl.ds(0, NTAIL)],
                        acc.at[pl.ds(16 * NPT, NTAIL)])
    plsc.subcore_barrier()

    def issue_idx(i, b):
        pltpu.async_copy(k_hbm.at[wid, i], KV[b], SI[b])
        pltpu.async_copy(d_hbm.at[wid, i], DV[b], SI[b])
        pltpu.async_copy(n_hbm.at[wid, i], NV[b], SI[b])

    def wait_idx(b):
        pltpu.make_async_copy(k_hbm.at[wid, 0], KV[b], SI[b]).wait()
        pltpu.make_async_copy(d_hbm.at[wid, 0], DV[b], SI[b]).wait()
        pltpu.make_async_copy(n_hbm.at[wid, 0], NV[b], SI[b]).wait()

    def issue_rows(b):
        pltpu.async_copy(tx_hbm.at[KV[b]], ROWS[b], SG[b])

    def wait_rows(b):
        pltpu.make_async_copy(tx_hbm.at[kv0], ROWS[b], SG[b]).wait()

    def scale(b):
        rows, nv = ROWS[b], NV[b]

        @plsc.parallel_loop(0, EC // 16, unroll=1)
        def _(gidx):
            v = nv[0, pl.ds(gidx * 16, 16)]
            for t in range(16):
                nb = jnp.full((16,), v[t], jnp.float32)
                e = gidx * 16 + t
                for j in range(8):
                    rows[e, pl.ds(j * 16, 16)] = rows[e, pl.ds(j * 16, 16)] * nb

    def wait_ss(b):
        pltpu.make_async_copy(ROWS[b], acc.at[DV[b]], SS[b]).wait()

    def body(i, b, first=False, last_g=False, last_i=False):
        bn = (b + 1) % 3
        bp = (b + 2) % 3
        if not last_g:
            wait_idx(bn)
            issue_rows(bn)
        wait_rows(b)
        scale(b)
        pltpu.async_copy(ROWS[b], acc.at[DV[b]], SS[b], add=True)
        if first:
            issue_idx(2, bp)
        elif not last_i:
            wait_ss(bp)
            issue_idx(i + 2, bp)
        else:
            wait_ss(bp)

    issue_idx(0, 0)
    issue_idx(1, 1)
    wait_idx(0)
    issue_rows(0)

    body(0, 0, first=True)
    body(1, 1)
    body(2, 2)

    def triple(t, _):
        i0 = 3 * t
        body(i0, 0)
        body(i0 + 1, 1)
        body(i0 + 2, 2)
        return 0
    lax.fori_loop(1, 41, triple, 0)

    body(123, 0, last_i=True)
    body(124, 1, last_g=True, last_i=True)
    wait_ss(1)
    plsc.subcore_barrier()

    pltpu.sync_copy(acc.at[pl.ds(s * NPT, NPT)],
                    out_hbm.at[c, pl.ds(s * NPT, NPT)])

    @pl.when(s == 15)
    def _():
        pltpu.sync_copy(acc.at[pl.ds(16 * NPT, NTAIL)],
                        out_hbm.at[c, pl.ds(16 * NPT, NTAIL)])


def _edge_agg(tx, k3, d3, n2):
    f = functools.partial(
        pl.kernel,
        mesh=_mesh(),
        out_type=jax.ShapeDtypeStruct((2, N, D), jnp.float32),
        scratch_types=(
            [pltpu.VMEM((EC,), jnp.int32)] * 6
            + [pltpu.VMEM((1, EC), jnp.float32)] * 3
            + [pltpu.VMEM((EC, D), jnp.float32)] * 3
            + [pltpu.VMEM_SHARED((N, D), jnp.float32)]
            + [pltpu.SemaphoreType.DMA] * 9
        ),
    )(_edge_agg_body)
    return f(tx, k3, d3, n2)


def _word_gather_body(wt_hbm, idx_hbm, out_hbm, iv, rows0, rows1, sg0, sg1):
    c = lax.axis_index("c")
    s = lax.axis_index("s")
    wid = s * 2 + c
    pltpu.sync_copy(idx_hbm.at[wid], iv)

    def issue(i, rows, sg):
        pltpu.async_copy(wt_hbm.at[iv.at[i]], rows, sg)

    def wait_in(rows, sg):
        pltpu.make_async_copy(wt_hbm.at[iv.at[0]], rows, sg).wait()

    def write(i, rows):
        pltpu.sync_copy(rows, out_hbm.at[pl.ds(wid * WPT + i * WC, WC)])

    issue(0, rows0, sg0)

    def pair(p, _):
        i0 = 2 * p
        wait_in(rows0, sg0)
        issue(i0 + 1, rows1, sg1)
        write(i0, rows0)
        wait_in(rows1, sg1)

        @pl.when(p < WCH // 2 - 1)
        def _():
            issue(i0 + 2, rows0, sg0)
        write(i0 + 1, rows1)
        return 0
    lax.fori_loop(0, WCH // 2, pair, 0)


def _word_gather(word_table, idx3):
    f = functools.partial(
        pl.kernel,
        mesh=_mesh(),
        out_type=jax.ShapeDtypeStruct((WG, D), jnp.float32),
        scratch_types=[
            pltpu.VMEM((WCH, WC), jnp.int32),
            pltpu.VMEM((WC, D), jnp.float32),
            pltpu.VMEM((WC, D), jnp.float32),
            pltpu.SemaphoreType.DMA,
            pltpu.SemaphoreType.DMA,
        ],
    )(_word_gather_body)
    return f(word_table, idx3)


def _tx_body(x_ref, wd_ref, out_ref):
    x = x_ref[...]
    for rr in range(R):
        out_ref[rr] = jnp.dot(x, wd_ref[rr], preferred_element_type=jnp.float32)


def _tx_compute(x, wd):
    nblk = 400
    return pl.pallas_call(
        _tx_body,
        grid=(N // nblk,),
        in_specs=[
            pl.BlockSpec((nblk, D), lambda n: (n, 0)),
            pl.BlockSpec((R, D, D), lambda n: (0, 0, 0)),
        ],
        out_specs=pl.BlockSpec((R, nblk, D), lambda n: (0, n, 0)),
        out_shape=jax.ShapeDtypeStruct((R, N, D), jnp.float32),
    )(x, wd)


def _node_body(x_ref, agg_ref, lw_ref, b_ref, out_ref):
    out_ref[...] = (agg_ref[0] + agg_ref[1] + b_ref[...]
                    + jnp.dot(x_ref[...], lw_ref[...],
                              preferred_element_type=jnp.float32))


def _node_out(x, agg2, loop_weight, h_bias):
    nblk = 400
    return pl.pallas_call(
        _node_body,
        grid=(N // nblk,),
        in_specs=[
            pl.BlockSpec((nblk, D), lambda n: (n, 0)),
            pl.BlockSpec((2, nblk, D), lambda n: (0, n, 0)),
            pl.BlockSpec((D, D), lambda n: (0, 0)),
            pl.BlockSpec((1, D), lambda n: (0, 0)),
        ],
        out_specs=pl.BlockSpec((nblk, D), lambda n: (n, 0)),
        out_shape=jax.ShapeDtypeStruct((N, D), jnp.float32),
    )(x, agg2, loop_weight, h_bias)


def _desc_body(xe_ref, w10, w20, w21, w30, w31, w32, b1, b2, b3, fcw, fcb,
               out_ref):
    bb = xe_ref.shape[0]
    xf = xe_ref[...].reshape(bb * LE, D)

    def tap(w):
        return jnp.dot(xf, w[...], preferred_element_type=jnp.float32).reshape(bb, LE, D)

    a10 = tap(w10)
    a20, a21 = tap(w20), tap(w21)
    a30, a31, a32 = tap(w30), tap(w31), tap(w32)
    f1 = jnp.max(jax.nn.relu(a10[:, :L] + b1[0]), axis=1)
    f2 = jnp.max(jax.nn.relu(a20[:, :L] + a21[:, 1:L + 1] + b2[0]), axis=1)
    f3 = jnp.max(jax.nn.relu(a30[:, :L] + a31[:, 1:L + 1] + a32[:, 2:L + 2]
                             + b3[0]), axis=1)
    allf = jnp.concatenate([f1, f2, f3], axis=1)
    out_ref[...] = jnp.dot(allf, fcw[...], preferred_element_type=jnp.float32) + fcb[...]


def _desc_compute(emb_ext, taps, biases, fcw_t, fc_b):
    bb = 64
    wspec = pl.BlockSpec((D, D), lambda n: (0, 0))
    bspec = pl.BlockSpec((1, D), lambda n: (0, 0))
    return pl.pallas_call(
        _desc_body,
        grid=(BD // bb,),
        in_specs=[pl.BlockSpec((bb, LE, D), lambda n: (n, 0, 0))]
                 + [wspec] * 6 + [bspec] * 3
                 + [pl.BlockSpec((3 * D, D), lambda n: (0, 0)), bspec],
        out_specs=pl.BlockSpec((bb, D), lambda n: (n, 0)),
        out_shape=jax.ShapeDtypeStruct((BD, D), jnp.float32),
    )(emb_ext, *taps, *biases, fcw_t, fc_b)


def kernel(g, h, r, norm, s_e_d_w_embeddings, entity_table, rgcn_weight,
           loop_weight, h_bias, word_table, conv_w1, conv_b1, conv_w2, conv_b2,
           conv_w3, conv_b3, fc_w, fc_b):
    x = entity_table

    wd = jnp.concatenate(
        [jnp.pad(rgcn_weight[:, b], ((0, 0), (0, 0), (b * BLK, D - (b + 1) * BLK)))
         for b in range(NB)], axis=1)

    wie = jnp.concatenate(
        [s_e_d_w_embeddings, s_e_d_w_embeddings[:, :2],
         jnp.zeros((BD, LE - L - 2), jnp.int32)], axis=1)
    idx3 = wie.reshape(NW, WCH, WC)
    emb = _word_gather(word_table, idx3).reshape(BD, LE, D)

    tx = _tx_compute(x, wd).reshape(R * N, D)

    kflat = (r * N + g[0]).astype(jnp.int32)
    k3 = kflat.reshape(NW, ECH, EC)
    d3 = g[1].reshape(NW, ECH, EC)
    n4 = norm.reshape(NW, ECH, 1, EC)

    agg2 = _edge_agg(tx, k3, d3, n4)
    node_out = _node_out(x, agg2, loop_weight, h_bias.reshape(1, D))

    taps = (conv_w1[:, :, 0].T,
            conv_w2[:, :, 0].T, conv_w2[:, :, 1].T,
            conv_w3[:, :, 0].T, conv_w3[:, :, 1].T, conv_w3[:, :, 2].T)
    biases = (conv_b1.reshape(1, D), conv_b2.reshape(1, D),
              conv_b3.reshape(1, D))
    desc = _desc_compute(emb, taps, biases, fc_w.T, fc_b.reshape(1, D))
    return node_out, desc


# speedup vs baseline: 1.3697x; 1.3697x over previous
import functools

import jax
import jax.numpy as jnp
from jax import lax
from jax.experimental import pallas as pl
from jax.experimental.pallas import tpu as pltpu
from jax.experimental.pallas import tpu_sc as plsc

N = 10000
E = 320000
D = 128
R = 20
NB = 4
BLK = 32
L = 100
BD = 1024
LE = 104

NW = 32
EPT = E // NW
EC = 80
ECH = EPT // EC
NPT = 624
NTAIL = N - 16 * NPT

WG = BD * LE
WPT = WG // NW
WC = 64
WCH = WPT // WC

_mesh = lambda: plsc.VectorSubcoreMesh(core_axis_name="c", subcore_axis_name="s")


def _edge_agg_body(tx_hbm, k_hbm, d_hbm, n_hbm, out_hbm,
                   kv0, kv1, kv2, dv0, dv1, dv2, nv0, nv1, nv2,
                   rows0, rows1, rows2, acc,
                   si0, si1, si2, sg0, sg1, sg2, ss0, ss1, ss2):
    c = lax.axis_index("c")
    s = lax.axis_index("s")
    wid = s * 2 + c
    KV = (kv0, kv1, kv2)
    DV = (dv0, dv1, dv2)
    NV = (nv0, nv1, nv2)
    ROWS = (rows0, rows1, rows2)
    SI = (si0, si1, si2)
    SG = (sg0, sg1, sg2)
    SS = (ss0, ss1, ss2)

    z16 = jnp.zeros((16,), jnp.float32)
    for j in range(8):
        def zrow(i, _, j=j):
            rows0[i, pl.ds(j * 16, 16)] = z16
            return 0
        lax.fori_loop(0, EC, zrow, 0)

    def zcp(i, _):
        pltpu.sync_copy(rows0, acc.at[pl.ds(s * NPT + i * EC, EC)])
        return 0
    lax.fori_loop(0, 7, zcp, 0)
    pltpu.sync_copy(rows0.at[pl.ds(0, 64)],
                    acc.at[pl.ds(s * NPT + 7 * EC, 64)])

    @pl.when(s == 15)
    def _():
        pltpu.sync_copy(rows0.at[pl.ds(0, NTAIL)],
                        acc.at[pl.ds(16 * NPT, NTAIL)])
    plsc.subcore_barrier()

    def issue_idx(i, b):
        pltpu.async_copy(k_hbm.at[wid, i], KV[b], SI[b])
        pltpu.async_copy(d_hbm.at[wid, i], DV[b], SI[b])
        pltpu.async_copy(n_hbm.at[wid, i], NV[b], SI[b])

    def wait_idx(b):
        pltpu.make_async_copy(k_hbm.at[wid, 0], KV[b], SI[b]).wait()
        pltpu.make_async_copy(d_hbm.at[wid, 0], DV[b], SI[b]).wait()
        pltpu.make_async_copy(n_hbm.at[wid, 0], NV[b], SI[b]).wait()

    def issue_rows(b):
        pltpu.async_copy(tx_hbm.at[KV[b]], ROWS[b], SG[b])

    def wait_rows(b):
        pltpu.make_async_copy(tx_hbm.at[kv0], ROWS[b], SG[b]).wait()

    def scale(b):
        rows, nv = ROWS[b], NV[b]

        @plsc.parallel_loop(0, EC // 16, unroll=1)
        def _(gidx):
            v = nv[0, pl.ds(gidx * 16, 16)]
            for t in range(16):
                nb = jnp.full((16,), v[t], jnp.float32)
                e = gidx * 16 + t
                for j in range(8):
                    rows[e, pl.ds(j * 16, 16)] = rows[e, pl.ds(j * 16, 16)] * nb

    def wait_ss(b):
        pltpu.make_async_copy(ROWS[b], acc.at[DV[b]], SS[b]).wait()

    def body(i, b, first=False, last_g=False, last_i=False):
        bn = (b + 1) % 3
        bp = (b + 2) % 3
        if not last_g:
            wait_idx(bn)
            issue_rows(bn)
        wait_rows(b)
        scale(b)
        pltpu.async_copy(ROWS[b], acc.at[DV[b]], SS[b], add=True)
        if first:
            issue_idx(2, bp)
        elif not last_i:
            wait_ss(bp)
            issue_idx(i + 2, bp)
        else:
            wait_ss(bp)

    issue_idx(0, 0)
    issue_idx(1, 1)
    wait_idx(0)
    issue_rows(0)

    body(0, 0, first=True)
    body(1, 1)
    body(2, 2)

    def triple(t, _):
        i0 = 3 * t
        body(i0, 0)
        body(i0 + 1, 1)
        body(i0 + 2, 2)
        return 0
    lax.fori_loop(1, 41, triple, 0)

    body(123, 0, last_i=True)
    body(124, 1, last_g=True, last_i=True)
    wait_ss(1)
    plsc.subcore_barrier()

    pltpu.sync_copy(acc.at[pl.ds(s * NPT, NPT)],
                    out_hbm.at[c, pl.ds(s * NPT, NPT)])

    @pl.when(s == 15)
    def _():
        pltpu.sync_copy(acc.at[pl.ds(16 * NPT, NTAIL)],
                        out_hbm.at[c, pl.ds(16 * NPT, NTAIL)])


def _edge_agg(tx, k3, d3, n2):
    f = functools.partial(
        pl.kernel,
        mesh=_mesh(),
        out_type=jax.ShapeDtypeStruct((2, N, D), jnp.float32),
        scratch_types=(
            [pltpu.VMEM((EC,), jnp.int32)] * 6
            + [pltpu.VMEM((1, EC), jnp.float32)] * 3
            + [pltpu.VMEM((EC, D), jnp.float32)] * 3
            + [pltpu.VMEM_SHARED((N, D), jnp.float32)]
            + [pltpu.SemaphoreType.DMA] * 9
        ),
    )(_edge_agg_body)
    return f(tx, k3, d3, n2)


def _word_gather_body(wt_hbm, idx_hbm, out_hbm, iv, rows0, rows1, sg0, sg1):
    c = lax.axis_index("c")
    s = lax.axis_index("s")
    wid = s * 2 + c
    pltpu.sync_copy(idx_hbm.at[wid], iv)

    def issue(i, rows, sg):
        pltpu.async_copy(wt_hbm.at[iv.at[i]], rows, sg)

    def wait_in(rows, sg):
        pltpu.make_async_copy(wt_hbm.at[iv.at[0]], rows, sg).wait()

    def write(i, rows):
        pltpu.sync_copy(rows, out_hbm.at[pl.ds(wid * WPT + i * WC, WC)])

    issue(0, rows0, sg0)

    def pair(p, _):
        i0 = 2 * p
        wait_in(rows0, sg0)
        issue(i0 + 1, rows1, sg1)
        write(i0, rows0)
        wait_in(rows1, sg1)

        @pl.when(p < WCH // 2 - 1)
        def _():
            issue(i0 + 2, rows0, sg0)
        write(i0 + 1, rows1)
        return 0
    lax.fori_loop(0, WCH // 2, pair, 0)


def _word_gather(word_table, idx3):
    f = functools.partial(
        pl.kernel,
        mesh=_mesh(),
        out_type=jax.ShapeDtypeStruct((WG, D), jnp.float32),
        scratch_types=[
            pltpu.VMEM((WCH, WC), jnp.int32),
            pltpu.VMEM((WC, D), jnp.float32),
            pltpu.VMEM((WC, D), jnp.float32),
            pltpu.SemaphoreType.DMA,
            pltpu.SemaphoreType.DMA,
        ],
    )(_word_gather_body)
    return f(word_table, idx3)


def _tx_body(x_ref, wd_ref, out_ref):
    x = x_ref[...]
    for rr in range(R):
        out_ref[rr] = jnp.dot(x, wd_ref[rr], preferred_element_type=jnp.float32)


def _tx_compute(x, wd):
    nblk = 400
    return pl.pallas_call(
        _tx_body,
        grid=(N // nblk,),
        in_specs=[
            pl.BlockSpec((nblk, D), lambda n: (n, 0)),
            pl.BlockSpec((R, D, D), lambda n: (0, 0, 0)),
        ],
        out_specs=pl.BlockSpec((R, nblk, D), lambda n: (0, n, 0)),
        out_shape=jax.ShapeDtypeStruct((R, N, D), jnp.float32),
    )(x, wd)


def _node_body(x_ref, agg_ref, lw_ref, b_ref, out_ref):
    out_ref[...] = (agg_ref[0] + agg_ref[1] + b_ref[...]
                    + jnp.dot(x_ref[...], lw_ref[...],
                              preferred_element_type=jnp.float32))


def _node_out(x, agg2, loop_weight, h_bias):
    nblk = 400
    return pl.pallas_call(
        _node_body,
        grid=(N // nblk,),
        in_specs=[
            pl.BlockSpec((nblk, D), lambda n: (n, 0)),
            pl.BlockSpec((2, nblk, D), lambda n: (0, n, 0)),
            pl.BlockSpec((D, D), lambda n: (0, 0)),
            pl.BlockSpec((1, D), lambda n: (0, 0)),
        ],
        out_specs=pl.BlockSpec((nblk, D), lambda n: (n, 0)),
        out_shape=jax.ShapeDtypeStruct((N, D), jnp.float32),
    )(x, agg2, loop_weight, h_bias)


def _desc_body(xe_ref, w10, w20, w21, w30, w31, w32, b1, b2, b3, fcw, fcb,
               out_ref):
    bb = xe_ref.shape[0]
    xf = xe_ref[...].reshape(bb * LE, D)

    def tap(w):
        return jnp.dot(xf, w[...], preferred_element_type=jnp.float32).reshape(bb, LE, D)

    a10 = tap(w10)
    a20, a21 = tap(w20), tap(w21)
    a30, a31, a32 = tap(w30), tap(w31), tap(w32)
    f1 = jnp.max(jax.nn.relu(a10[:, :L] + b1[0]), axis=1)
    f2 = jnp.max(jax.nn.relu(a20[:, :L] + a21[:, 1:L + 1] + b2[0]), axis=1)
    f3 = jnp.max(jax.nn.relu(a30[:, :L] + a31[:, 1:L + 1] + a32[:, 2:L + 2]
                             + b3[0]), axis=1)
    allf = jnp.concatenate([f1, f2, f3], axis=1)
    out_ref[...] = jnp.dot(allf, fcw[...], preferred_element_type=jnp.float32) + fcb[...]


def _desc_compute(emb_ext, taps, biases, fcw_t, fc_b):
    bb = 64
    wspec = pl.BlockSpec((D, D), lambda n: (0, 0))
    bspec = pl.BlockSpec((1, D), lambda n: (0, 0))
    return pl.pallas_call(
        _desc_body,
        grid=(BD // bb,),
        in_specs=[pl.BlockSpec((bb, LE, D), lambda n: (n, 0, 0))]
                 + [wspec] * 6 + [bspec] * 3
                 + [pl.BlockSpec((3 * D, D), lambda n: (0, 0)), bspec],
        out_specs=pl.BlockSpec((bb, D), lambda n: (n, 0)),
        out_shape=jax.ShapeDtypeStruct((BD, D), jnp.float32),
    )(emb_ext, *taps, *biases, fcw_t, fc_b)


def kernel(g, h, r, norm, s_e_d_w_embeddings, entity_table, rgcn_weight,
           loop_weight, h_bias, word_table, conv_w1, conv_b1, conv_w2, conv_b2,
           conv_w3, conv_b3, fc_w, fc_b):
    x = entity_table

    wd = jnp.concatenate(
        [jnp.pad(rgcn_weight[:, b], ((0, 0), (0, 0), (b * BLK, D - (b + 1) * BLK)))
         for b in range(NB)], axis=1)

    wie = jnp.concatenate(
        [s_e_d_w_embeddings, s_e_d_w_embeddings[:, :LE - L]], axis=1)
    idx3 = wie.reshape(NW, WCH, WC)
    emb = _word_gather(word_table, idx3).reshape(BD, LE, D)

    tx = _tx_compute(x, wd).reshape(R * N, D)

    kflat = (r * N + g[0]).astype(jnp.int32)
    k3 = kflat.reshape(NW, ECH, EC)
    d3 = g[1].reshape(NW, ECH, EC)
    n4 = norm.reshape(NW, ECH, 1, EC)

    agg2 = _edge_agg(tx, k3, d3, n4)
    node_out = _node_out(x, agg2, loop_weight, h_bias.reshape(1, D))

    taps = (conv_w1[:, :, 0].T,
            conv_w2[:, :, 0].T, conv_w2[:, :, 1].T,
            conv_w3[:, :, 0].T, conv_w3[:, :, 1].T, conv_w3[:, :, 2].T)
    biases = (conv_b1.reshape(1, D), conv_b2.reshape(1, D),
              conv_b3.reshape(1, D))
    desc = _desc_compute(emb, taps, biases, fc_w.T, fc_b.reshape(1, D))
    return node_out, desc
